# baseline (speedup 1.0000x reference)
import functools

import jax
import jax.numpy as jnp
from jax import lax
from jax.experimental import pallas as pl
from jax.experimental.pallas import tpu as pltpu
from jax.experimental.pallas import tpu_sc as plsc

N = 10000
E = 320000
D = 128
H = 128
STEPS = 4
G = 64

ROW_BLK = 1000
N_BLKS = N // ROW_BLK

NC = 2
NS = 16
NW = NC * NS
CHUNK = 128
CHUNKS_PER_TILE = 80
E_PAD = NW * CHUNKS_PER_TILE * CHUNK
AGG_ROWS = 10240
ROWS_PER_TILE = AGG_ROWS // NS


def _embed_body(x_ref, win_ref, wc_ref, whht_ref, bhh_ref,
                h_ref, m_ref, gh_ref):
    h = jnp.tanh(jnp.dot(x_ref[...], win_ref[...],
                         preferred_element_type=jnp.float32))
    h_ref[...] = h
    m_ref[...] = jnp.dot(h, wc_ref[...], preferred_element_type=jnp.float32)
    gh_ref[...] = jnp.dot(h, whht_ref[...],
                          preferred_element_type=jnp.float32) + bhh_ref[...]


def _embed(x, w_in, wc0, w_hht, b_hh2):
    return pl.pallas_call(
        _embed_body,
        grid=(N_BLKS,),
        in_specs=[
            pl.BlockSpec((ROW_BLK, D), lambda i: (i, 0)),
            pl.BlockSpec((D, H), lambda i: (0, 0)),
            pl.BlockSpec((H, H), lambda i: (0, 0)),
            pl.BlockSpec((H, 3 * H), lambda i: (0, 0)),
            pl.BlockSpec((1, 3 * H), lambda i: (0, 0)),
        ],
        out_specs=[
            pl.BlockSpec((ROW_BLK, H), lambda i: (i, 0)),
            pl.BlockSpec((ROW_BLK, H), lambda i: (i, 0)),
            pl.BlockSpec((ROW_BLK, 3 * H), lambda i: (i, 0)),
        ],
        out_shape=[
            jax.ShapeDtypeStruct((N, H), jnp.float32),
            jax.ShapeDtypeStruct((N, H), jnp.float32),
            jax.ShapeDtypeStruct((N, 3 * H), jnp.float32),
        ],
    )(x, w_in, wc0, w_hht, b_hh2)


def _unpack_chunk(packed_v, t, sb, db):
    for j in range(CHUNK // 16):
        p = packed_v[t, pl.ds(j * 16, 16)]
        sb[pl.ds(j * 16, 16)] = p & 0xFFFF
        db[pl.ds(j * 16, 16)] = lax.shift_right_logical(p, 16)


def _aggregate_body(m_hbm, packed_hbm, zeros_hbm, out_hbm,
                    packed_v, sb0, db0, sb1, db1, rows_v, agg_s,
                    gsem0, gsem1, ssem0, ssem1):
    c = lax.axis_index("c")
    s = lax.axis_index("s")
    row0 = s * ROWS_PER_TILE
    pltpu.sync_copy(zeros_hbm.at[pl.ds(row0, ROWS_PER_TILE)],
                    agg_s.at[pl.ds(row0, ROWS_PER_TILE)])
    pltpu.sync_copy(packed_hbm.at[c].at[s], packed_v)
    plsc.subcore_barrier()

    rows0 = rows_v.at[0]
    rows1 = rows_v.at[1]
    nt2 = CHUNKS_PER_TILE // 2

    _unpack_chunk(packed_v, 0, sb0, db0)
    pltpu.async_copy(m_hbm.at[sb0], rows0, gsem0)

    def body(t2, carry):
        a = t2 * 2
        b = a + 1
        pltpu.make_async_copy(m_hbm.at[sb0], rows0, gsem0).wait()
        pltpu.async_copy(rows0, agg_s.at[db0], ssem0, add=True)

        @pl.when(t2 > 0)
        def _():
            pltpu.make_async_copy(rows1, agg_s.at[db1], ssem1).wait()
        _unpack_chunk(packed_v, b, sb1, db1)
        pltpu.async_copy(m_hbm.at[sb1], rows1, gsem1)

        pltpu.make_async_copy(m_hbm.at[sb1], rows1, gsem1).wait()
        pltpu.async_copy(rows1, agg_s.at[db1], ssem1, add=True)
        pltpu.make_async_copy(rows0, agg_s.at[db0], ssem0).wait()

        @pl.when(t2 + 1 < nt2)
        def _():
            _unpack_chunk(packed_v, a + 2, sb0, db0)
            pltpu.async_copy(m_hbm.at[sb0], rows0, gsem0)
        return carry

    lax.fori_loop(0, nt2, body, 0)
    pltpu.make_async_copy(rows1, agg_s.at[db1], ssem1).wait()
    plsc.subcore_barrier()
    pltpu.sync_copy(agg_s.at[pl.ds(row0, ROWS_PER_TILE)],
                    out_hbm.at[c].at[pl.ds(row0, ROWS_PER_TILE)])


@functools.lru_cache(maxsize=1)
def _build_aggregate():
    mesh = plsc.VectorSubcoreMesh(core_axis_name="c", subcore_axis_name="s",
                                  num_cores=NC, num_subcores=NS)
    return pl.kernel(
        _aggregate_body,
        out_type=jax.ShapeDtypeStruct((2, AGG_ROWS, H), jnp.float32),
        mesh=mesh,
        scratch_types=[
            pltpu.VMEM((CHUNKS_PER_TILE, CHUNK), jnp.int32),
            pltpu.VMEM((CHUNK,), jnp.int32),
            pltpu.VMEM((CHUNK,), jnp.int32),
            pltpu.VMEM((CHUNK,), jnp.int32),
            pltpu.VMEM((CHUNK,), jnp.int32),
            pltpu.VMEM((2, CHUNK, H), jnp.float32),
            pltpu.VMEM_SHARED((AGG_ROWS, H), jnp.float32),
            pltpu.SemaphoreType.DMA,
            pltpu.SemaphoreType.DMA,
            pltpu.SemaphoreType.DMA,
            pltpu.SemaphoreType.DMA,
        ],
    )


def _aggregate(m, packed_slab, zeros):
    return _build_aggregate()(m, packed_slab, zeros)


def _gru_math(h_ref, agg_ref, gh_ref, wiht_ref, bih_ref):
    a = agg_ref[0] + agg_ref[1]
    gi = jnp.dot(a, wiht_ref[...], preferred_element_type=jnp.float32) \
        + bih_ref[...]
    gh = gh_ref[...]
    h = h_ref[...]
    r = jax.nn.sigmoid(gi[:, :H] + gh[:, :H])
    z = jax.nn.sigmoid(gi[:, H:2 * H] + gh[:, H:2 * H])
    n = jnp.tanh(gi[:, 2 * H:] + r * gh[:, 2 * H:])
    return (1.0 - z) * n + z * h


def _gru_body(h_ref, agg_ref, gh_ref, wiht_ref, bih_ref, wc_ref, whht_ref,
              bhh_ref, hn_ref, m_ref, ghn_ref):
    h_new = _gru_math(h_ref, agg_ref, gh_ref, wiht_ref, bih_ref)
    hn_ref[...] = h_new
    m_ref[...] = jnp.dot(h_new, wc_ref[...],
                         preferred_element_type=jnp.float32)
    ghn_ref[...] = jnp.dot(h_new, whht_ref[...],
                           preferred_element_type=jnp.float32) + bhh_ref[...]


def _gru_project(h, agg, gh, w_iht, b_ih2, wc, w_hht, b_hh2):
    return pl.pallas_call(
        _gru_body,
        grid=(N_BLKS,),
        in_specs=[
            pl.BlockSpec((ROW_BLK, H), lambda i: (i, 0)),
            pl.BlockSpec((2, ROW_BLK, H), lambda i: (0, i, 0)),
            pl.BlockSpec((ROW_BLK, 3 * H), lambda i: (i, 0)),
            pl.BlockSpec((H, 3 * H), lambda i: (0, 0)),
            pl.BlockSpec((1, 3 * H), lambda i: (0, 0)),
            pl.BlockSpec((H, H), lambda i: (0, 0)),
            pl.BlockSpec((H, 3 * H), lambda i: (0, 0)),
            pl.BlockSpec((1, 3 * H), lambda i: (0, 0)),
        ],
        out_specs=[
            pl.BlockSpec((ROW_BLK, H), lambda i: (i, 0)),
            pl.BlockSpec((ROW_BLK, H), lambda i: (i, 0)),
            pl.BlockSpec((ROW_BLK, 3 * H), lambda i: (i, 0)),
        ],
        out_shape=[
            jax.ShapeDtypeStruct((N, H), jnp.float32),
            jax.ShapeDtypeStruct((N, H), jnp.float32),
            jax.ShapeDtypeStruct((N, 3 * H), jnp.float32),
        ],
    )(h, agg, gh, w_iht, b_ih2, wc, w_hht, b_hh2)


def _pool_body(h_ref, agg_ref, gh_ref, wiht_ref, bih_ref, batch_ref,
               wp_ref, bp_ref, out_ref, sums_ref, cnt_ref):
    i = pl.program_id(0)
    h_new = _gru_math(h_ref, agg_ref, gh_ref, wiht_ref, bih_ref)
    b = batch_ref[0, 0, :]
    onehot = (b[:, None] ==
              lax.broadcasted_iota(jnp.int32, (ROW_BLK, G), 1)
              ).astype(jnp.float32)

    @pl.when(i == 0)
    def _init():
        sums_ref[...] = jnp.zeros_like(sums_ref)
        cnt_ref[...] = jnp.zeros_like(cnt_ref)

    sums_ref[...] += lax.dot_general(
        onehot, h_new, (((0,), (0,)), ((), ())),
        preferred_element_type=jnp.float32)
    cnt_ref[0, :] += jnp.sum(onehot, axis=0)

    @pl.when(i == N_BLKS - 1)
    def _final():
        pooled = sums_ref[...] / jnp.maximum(cnt_ref[0, :], 1.0)[:, None]
        pooled = jnp.maximum(pooled, 0.0)
        out_ref[...] = jnp.dot(pooled, wp_ref[...],
                               preferred_element_type=jnp.float32) \
            + bp_ref[...]


def _gru_pool(h, agg, gh, w_iht, b_ih2, batch3, w_pred, b_pred2):
    return pl.pallas_call(
        _pool_body,
        grid=(N_BLKS,),
        in_specs=[
            pl.BlockSpec((ROW_BLK, H), lambda i: (i, 0)),
            pl.BlockSpec((2, ROW_BLK, H), lambda i: (0, i, 0)),
            pl.BlockSpec((ROW_BLK, 3 * H), lambda i: (i, 0)),
            pl.BlockSpec((H, 3 * H), lambda i: (0, 0)),
            pl.BlockSpec((1, 3 * H), lambda i: (0, 0)),
            pl.BlockSpec((1, 1, ROW_BLK), lambda i: (i, 0, 0)),
            pl.BlockSpec((H, 1), lambda i: (0, 0)),
            pl.BlockSpec((1, 1), lambda i: (0, 0)),
        ],
        out_specs=[pl.BlockSpec((G, 1), lambda i: (0, 0))],
        out_shape=[jax.ShapeDtypeStruct((G, 1), jnp.float32)],
        scratch_shapes=[
            pltpu.VMEM((G, H), jnp.float32),
            pltpu.VMEM((8, G), jnp.float32),
        ],
    )(h, agg, gh, w_iht, b_ih2, batch3, w_pred, b_pred2)[0]


def kernel(x, edge_index, batch, W_in, W_conv, w_ih, w_hh, b_ih, b_hh,
           W_pred, b_pred):
    src = edge_index[0]
    dst = edge_index[1]
    pad = E_PAD - E
    packed = src | (dst << 16)
    packed_slab = jnp.concatenate(
        [packed, jnp.full((pad,), N << 16, jnp.int32)]
    ).reshape(NC, NS, CHUNKS_PER_TILE, CHUNK)
    zeros = jnp.zeros((AGG_ROWS, H), jnp.float32)
    batch3 = batch.reshape(N_BLKS, 1, ROW_BLK)
    w_iht = w_ih.T
    w_hht = w_hh.T
    b_ih2 = b_ih.reshape(1, 3 * H)
    b_hh2 = b_hh.reshape(1, 3 * H)
    b_pred2 = b_pred.reshape(1, 1)

    h, m, gh = _embed(x, W_in, W_conv[0], w_hht, b_hh2)
    for i in range(STEPS):
        agg = _aggregate(m, packed_slab, zeros)
        if i < STEPS - 1:
            h, m, gh = _gru_project(h, agg, gh, w_iht, b_ih2,
                                    W_conv[i + 1], w_hht, b_hh2)
        else:
            out = _gru_pool(h, agg, gh, w_iht, b_ih2, batch3,
                            W_pred, b_pred2)
    return out.reshape(G)


# speedup vs baseline: 1.6118x; 1.6118x over previous
import functools

import jax
import jax.numpy as jnp
from jax import lax
from jax.experimental import pallas as pl
from jax.experimental.pallas import tpu as pltpu
from jax.experimental.pallas import tpu_sc as plsc

N = 10000
E = 320000
D = 128
H = 128
STEPS = 4
G = 64

ROW_BLK = 1000
N_BLKS = N // ROW_BLK

NC = 2
NS = 16
NW = NC * NS
CHUNK = 128
CPT0 = 58
CPT1 = 99
CHUNKS_PER_TILE = max(CPT0, CPT1)
E_PAD = NS * (CPT0 + CPT1) * CHUNK
AGG_ROWS = 10240
ROWS_PER_TILE = AGG_ROWS // NS


def _embed_body(x_ref, win_ref, wc_ref, whht_ref, bhh_ref,
                h_ref, m_ref, gh_ref):
    h = jnp.tanh(jnp.dot(x_ref[...], win_ref[...],
                         preferred_element_type=jnp.float32))
    h_ref[...] = h
    m_ref[...] = jnp.dot(h, wc_ref[...], preferred_element_type=jnp.float32)
    gh_ref[...] = jnp.dot(h, whht_ref[...],
                          preferred_element_type=jnp.float32) + bhh_ref[...]


def _embed(x, w_in, wc0, w_hht, b_hh2):
    return pl.pallas_call(
        _embed_body,
        grid=(N_BLKS,),
        in_specs=[
            pl.BlockSpec((ROW_BLK, D), lambda i: (i, 0)),
            pl.BlockSpec((D, H), lambda i: (0, 0)),
            pl.BlockSpec((H, H), lambda i: (0, 0)),
            pl.BlockSpec((H, 3 * H), lambda i: (0, 0)),
            pl.BlockSpec((1, 3 * H), lambda i: (0, 0)),
        ],
        out_specs=[
            pl.BlockSpec((ROW_BLK, H), lambda i: (i, 0)),
            pl.BlockSpec((ROW_BLK, H), lambda i: (i, 0)),
            pl.BlockSpec((ROW_BLK, 3 * H), lambda i: (i, 0)),
        ],
        out_shape=[
            jax.ShapeDtypeStruct((N, H), jnp.float32),
            jax.ShapeDtypeStruct((N, H), jnp.float32),
            jax.ShapeDtypeStruct((N, 3 * H), jnp.float32),
        ],
    )(x, w_in, wc0, w_hht, b_hh2)


def _aggregate_body(m_hbm, src_hbm, dst_hbm, zeros_hbm, out_hbm,
                    src_v, dst_v, rows_v, agg_s, gsem, ssem):
    c = lax.axis_index("c")
    s = lax.axis_index("s")
    row0 = s * ROWS_PER_TILE
    pltpu.sync_copy(zeros_hbm.at[pl.ds(row0, ROWS_PER_TILE)],
                    agg_s.at[pl.ds(row0, ROWS_PER_TILE)])
    pltpu.sync_copy(src_hbm.at[c].at[s], src_v)
    pltpu.sync_copy(dst_hbm.at[c].at[s], dst_v)
    plsc.subcore_barrier()

    def body(t, carry):
        pltpu.async_copy(m_hbm.at[src_v.at[t]], rows_v, gsem).wait()
        pltpu.async_copy(rows_v, agg_s.at[dst_v.at[t]], ssem, add=True).wait()
        return carry

    nt = jnp.where(c == 0, CPT0, CPT1)
    lax.fori_loop(0, nt, body, 0)
    plsc.subcore_barrier()
    pltpu.sync_copy(agg_s.at[pl.ds(row0, ROWS_PER_TILE)],
                    out_hbm.at[c].at[pl.ds(row0, ROWS_PER_TILE)])


@functools.lru_cache(maxsize=1)
def _build_aggregate():
    mesh = plsc.VectorSubcoreMesh(core_axis_name="c", subcore_axis_name="s",
                                  num_cores=NC, num_subcores=NS)
    return pl.kernel(
        _aggregate_body,
        out_type=jax.ShapeDtypeStruct((2, AGG_ROWS, H), jnp.float32),
        mesh=mesh,
        scratch_types=[
            pltpu.VMEM((CHUNKS_PER_TILE, CHUNK), jnp.int32),
            pltpu.VMEM((CHUNKS_PER_TILE, CHUNK), jnp.int32),
            pltpu.VMEM((CHUNK, H), jnp.float32),
            pltpu.VMEM_SHARED((AGG_ROWS, H), jnp.float32),
            pltpu.SemaphoreType.DMA,
            pltpu.SemaphoreType.DMA,
        ],
    )


def _aggregate(m, src_slab, dst_slab, zeros):
    return _build_aggregate()(m, src_slab, dst_slab, zeros)


def _gru_math(h_ref, agg_ref, gh_ref, wiht_ref, bih_ref):
    a = agg_ref[0] + agg_ref[1]
    gi = jnp.dot(a, wiht_ref[...], preferred_element_type=jnp.float32) \
        + bih_ref[...]
    gh = gh_ref[...]
    h = h_ref[...]
    r = jax.nn.sigmoid(gi[:, :H] + gh[:, :H])
    z = jax.nn.sigmoid(gi[:, H:2 * H] + gh[:, H:2 * H])
    n = jnp.tanh(gi[:, 2 * H:] + r * gh[:, 2 * H:])
    return (1.0 - z) * n + z * h


def _gru_body(h_ref, agg_ref, gh_ref, wiht_ref, bih_ref, wc_ref, whht_ref,
              bhh_ref, hn_ref, m_ref, ghn_ref):
    h_new = _gru_math(h_ref, agg_ref, gh_ref, wiht_ref, bih_ref)
    hn_ref[...] = h_new
    m_ref[...] = jnp.dot(h_new, wc_ref[...],
                         preferred_element_type=jnp.float32)
    ghn_ref[...] = jnp.dot(h_new, whht_ref[...],
                           preferred_element_type=jnp.float32) + bhh_ref[...]


def _gru_project(h, agg, gh, w_iht, b_ih2, wc, w_hht, b_hh2):
    return pl.pallas_call(
        _gru_body,
        grid=(N_BLKS,),
        in_specs=[
            pl.BlockSpec((ROW_BLK, H), lambda i: (i, 0)),
            pl.BlockSpec((2, ROW_BLK, H), lambda i: (0, i, 0)),
            pl.BlockSpec((ROW_BLK, 3 * H), lambda i: (i, 0)),
            pl.BlockSpec((H, 3 * H), lambda i: (0, 0)),
            pl.BlockSpec((1, 3 * H), lambda i: (0, 0)),
            pl.BlockSpec((H, H), lambda i: (0, 0)),
            pl.BlockSpec((H, 3 * H), lambda i: (0, 0)),
            pl.BlockSpec((1, 3 * H), lambda i: (0, 0)),
        ],
        out_specs=[
            pl.BlockSpec((ROW_BLK, H), lambda i: (i, 0)),
            pl.BlockSpec((ROW_BLK, H), lambda i: (i, 0)),
            pl.BlockSpec((ROW_BLK, 3 * H), lambda i: (i, 0)),
        ],
        out_shape=[
            jax.ShapeDtypeStruct((N, H), jnp.float32),
            jax.ShapeDtypeStruct((N, H), jnp.float32),
            jax.ShapeDtypeStruct((N, 3 * H), jnp.float32),
        ],
    )(h, agg, gh, w_iht, b_ih2, wc, w_hht, b_hh2)


def _pool_body(h_ref, agg_ref, gh_ref, wiht_ref, bih_ref, batch_ref,
               wp_ref, bp_ref, out_ref, sums_ref, cnt_ref):
    i = pl.program_id(0)
    h_new = _gru_math(h_ref, agg_ref, gh_ref, wiht_ref, bih_ref)
    b = batch_ref[0, 0, :]
    onehot = (b[:, None] ==
              lax.broadcasted_iota(jnp.int32, (ROW_BLK, G), 1)
              ).astype(jnp.float32)

    @pl.when(i == 0)
    def _init():
        sums_ref[...] = jnp.zeros_like(sums_ref)
        cnt_ref[...] = jnp.zeros_like(cnt_ref)

    sums_ref[...] += lax.dot_general(
        onehot, h_new, (((0,), (0,)), ((), ())),
        preferred_element_type=jnp.float32)
    cnt_ref[0, :] += jnp.sum(onehot, axis=0)

    @pl.when(i == N_BLKS - 1)
    def _final():
        pooled = sums_ref[...] / jnp.maximum(cnt_ref[0, :], 1.0)[:, None]
        pooled = jnp.maximum(pooled, 0.0)
        out_ref[...] = jnp.dot(pooled, wp_ref[...],
                               preferred_element_type=jnp.float32) \
            + bp_ref[...]


def _gru_pool(h, agg, gh, w_iht, b_ih2, batch3, w_pred, b_pred2):
    return pl.pallas_call(
        _pool_body,
        grid=(N_BLKS,),
        in_specs=[
            pl.BlockSpec((ROW_BLK, H), lambda i: (i, 0)),
            pl.BlockSpec((2, ROW_BLK, H), lambda i: (0, i, 0)),
            pl.BlockSpec((ROW_BLK, 3 * H), lambda i: (i, 0)),
            pl.BlockSpec((H, 3 * H), lambda i: (0, 0)),
            pl.BlockSpec((1, 3 * H), lambda i: (0, 0)),
            pl.BlockSpec((1, 1, ROW_BLK), lambda i: (i, 0, 0)),
            pl.BlockSpec((H, 1), lambda i: (0, 0)),
            pl.BlockSpec((1, 1), lambda i: (0, 0)),
        ],
        out_specs=[pl.BlockSpec((G, 1), lambda i: (0, 0))],
        out_shape=[jax.ShapeDtypeStruct((G, 1), jnp.float32)],
        scratch_shapes=[
            pltpu.VMEM((G, H), jnp.float32),
            pltpu.VMEM((8, G), jnp.float32),
        ],
    )(h, agg, gh, w_iht, b_ih2, batch3, w_pred, b_pred2)[0]


def kernel(x, edge_index, batch, W_in, W_conv, w_ih, w_hh, b_ih, b_hh,
           W_pred, b_pred):
    src = edge_index[0]
    dst = edge_index[1]
    pad = E_PAD - E
    def _slabify(v, fill):
        vp = jnp.concatenate([v, jnp.full((pad,), fill, jnp.int32)])
        e0 = NS * CPT0 * CHUNK
        s0 = vp[:e0].reshape(NS, CPT0, CHUNK)
        s1 = vp[e0:].reshape(NS, CPT1, CHUNK)
        s0 = jnp.pad(s0, ((0, 0), (0, CHUNKS_PER_TILE - CPT0), (0, 0)),
                     constant_values=fill)
        s1 = jnp.pad(s1, ((0, 0), (0, CHUNKS_PER_TILE - CPT1), (0, 0)),
                     constant_values=fill)
        return jnp.stack([s0, s1], axis=0)

    src_slab = _slabify(src, 0)
    dst_slab = _slabify(dst, N)
    zeros = jnp.zeros((AGG_ROWS, H), jnp.float32)
    batch3 = batch.reshape(N_BLKS, 1, ROW_BLK)
    w_iht = w_ih.T
    w_hht = w_hh.T
    b_ih2 = b_ih.reshape(1, 3 * H)
    b_hh2 = b_hh.reshape(1, 3 * H)
    b_pred2 = b_pred.reshape(1, 1)

    h, m, gh = _embed(x, W_in, W_conv[0], w_hht, b_hh2)
    for i in range(STEPS):
        agg = _aggregate(m, src_slab, dst_slab, zeros)
        if i < STEPS - 1:
            h, m, gh = _gru_project(h, agg, gh, w_iht, b_ih2,
                                    W_conv[i + 1], w_hht, b_hh2)
        else:
            out = _gru_pool(h, agg, gh, w_iht, b_ih2, batch3,
                            W_pred, b_pred2)
    return out.reshape(G)
